# baseline (speedup 1.0000x reference)
import functools

import jax
import jax.numpy as jnp
from jax import lax
from jax.experimental import pallas as pl
from jax.experimental.pallas import tpu as pltpu
from jax.experimental.pallas import tpu_sc as plsc

UPSAMPLE = 4
V = 100000
H = 64
B, Q, T = 1024, 8, 200
TOUT = T * UPSAMPLE

RS_BLK = 5120
RS_GRID = -(-V // RS_BLK)

NW = 32
TT = 8
BB = 128
NTASK = (T // TT) * (B // BB)
NCB = B // BB
TPW = 7


def _rowsum_body(x_ref, o_ref):
    o_ref[...] = jnp.sum(x_ref[...], axis=0)


def _rowsum_tc(table):
    return pl.pallas_call(
        _rowsum_body,
        grid=(RS_GRID,),
        in_specs=[pl.BlockSpec((H, RS_BLK), lambda i: (0, i))],
        out_specs=pl.BlockSpec((RS_BLK,), lambda i: (i,)),
        out_shape=jax.ShapeDtypeStruct((V,), jnp.float32),
    )(table.T)


def _sc_body(codes_hbm, rowsum_hbm, out_hbm, rowsum_v, codes_v, out_v,
             rowsum_sh, rs_sem, c_sems, o_sems):
    cid = lax.axis_index("c")
    sid = lax.axis_index("s")
    wid = sid * 2 + cid

    tk0 = (wid * (NTASK - TPW)) // (NW - 1)
    tbs = []
    cbs = []
    for j in range(TPW):
        tk = tk0 + j
        tbs.append(tk // NCB)
        cbs.append(tk % NCB)

    def codes_dma(j):
        return pltpu.async_copy(
            codes_hbm.at[:, pl.ds(tbs[j] * TT, TT), pl.ds(cbs[j] * BB, BB)],
            codes_v.at[j % 2],
            c_sems[j % 2],
        )

    c_hs = {0: codes_dma(0)}

    @pl.when(sid == 0)
    def _stage_shared():
        pltpu.sync_copy(rowsum_hbm, rowsum_sh)

    plsc.subcore_barrier()
    rs_h = pltpu.async_copy(rowsum_sh, rowsum_v, rs_sem)
    rs_h.wait()

    o_hs = {}
    for j in range(TPW):
        if j + 1 < TPW:
            c_hs[j + 1] = codes_dma(j + 1)
        c_hs[j].wait()
        if j >= 2:
            o_hs[j - 2].wait()

        def bg_body(bg, carry2):
            b16 = bg * 16
            for tt in range(TT):
                g = [
                    plsc.load_gather(
                        rowsum_v, [codes_v[j % 2, q, tt, pl.ds(b16, 16)]]
                    )
                    for q in range(Q)
                ]
                while len(g) > 1:
                    g = [a + b for a, b in zip(g[::2], g[1::2])]
                off = ((tbs[j] * TT + tt) * UPSAMPLE) * 0.001
                val = g[0] + off
                for k in range(UPSAMPLE):
                    out_v[j % 2, UPSAMPLE * tt + k, pl.ds(b16, 16)] = val
                    if k + 1 < UPSAMPLE:
                        val = val + 0.001
            return carry2

        lax.fori_loop(0, BB // 16, bg_body, 0)
        o_hs[j] = pltpu.async_copy(
            out_v.at[j % 2],
            out_hbm.at[pl.ds(tbs[j] * TT * UPSAMPLE, TT * UPSAMPLE),
                       pl.ds(cbs[j] * BB, BB)],
            o_sems[j % 2],
        )
    o_hs[TPW - 2].wait()
    o_hs[TPW - 1].wait()


@functools.partial(
    pl.kernel,
    out_type=jax.ShapeDtypeStruct((TOUT, B), jnp.float32),
    mesh=plsc.VectorSubcoreMesh(core_axis_name="c", subcore_axis_name="s"),
    scratch_types=[
        pltpu.VMEM((V,), jnp.float32),
        pltpu.VMEM((2, Q, TT, BB), jnp.int32),
        pltpu.VMEM((2, UPSAMPLE * TT, BB), jnp.float32),
        pltpu.VMEM_SHARED((V,), jnp.float32),
        pltpu.SemaphoreType.DMA,
        pltpu.SemaphoreType.DMA,
        pltpu.SemaphoreType.DMA,
        pltpu.SemaphoreType.DMA,
        pltpu.SemaphoreType.DMA,
    ],
    compiler_params=pltpu.CompilerParams(needs_layout_passes=False),
)
def _sc_gather(codes_hbm, rowsum_hbm, out_hbm, rowsum_v, codes_v, out_v,
               rowsum_sh, rs_sem, c_sem0, c_sem1, o_sem0, o_sem1):
    _sc_body(codes_hbm, rowsum_hbm, out_hbm, rowsum_v, codes_v, out_v,
             rowsum_sh, rs_sem, [c_sem0, c_sem1], [o_sem0, o_sem1])


def kernel(codes, table):
    rowsum = _rowsum_tc(table)
    out_t = _sc_gather(codes.transpose(1, 2, 0), rowsum)
    return out_t.T.reshape(B, 1, TOUT)


# speedup vs baseline: 1.1571x; 1.1571x over previous
import functools

import jax
import jax.numpy as jnp
from jax import lax
from jax.experimental import pallas as pl
from jax.experimental.pallas import tpu as pltpu
from jax.experimental.pallas import tpu_sc as plsc

UPSAMPLE = 4
V = 100000
H = 64
B, Q, T = 1024, 8, 200
TOUT = T * UPSAMPLE

RS_BLK = 20480
RS_GRID = -(-V // RS_BLK)

NW = 32
TT = 8
BB = 128
NTASK = (T // TT) * (B // BB)
NCB = B // BB
TPW = 7


def _rowsum_body(x_ref, o_ref):
    o_ref[...] = jnp.sum(x_ref[...], axis=0)


def _rowsum_tc(table):
    return pl.pallas_call(
        _rowsum_body,
        grid=(RS_GRID,),
        in_specs=[pl.BlockSpec((H, RS_BLK), lambda i: (0, i))],
        out_specs=pl.BlockSpec((RS_BLK,), lambda i: (i,)),
        out_shape=jax.ShapeDtypeStruct((V,), jnp.float32),
    )(table.T)


def _sc_body(codes_hbm, rowsum_hbm, out_hbm, rowsum_v, codes_v, out_v,
             rowsum_sh, rs_sem, c_sems, o_sems):
    cid = lax.axis_index("c")
    sid = lax.axis_index("s")
    wid = sid * 2 + cid

    tk0 = (wid * (NTASK - TPW)) // (NW - 1)
    tbs = []
    cbs = []
    for j in range(TPW):
        tk = tk0 + j
        tbs.append(tk // NCB)
        cbs.append(tk % NCB)

    def codes_dma(j):
        return pltpu.async_copy(
            codes_hbm.at[:, pl.ds(tbs[j] * TT, TT), pl.ds(cbs[j] * BB, BB)],
            codes_v.at[j % 2],
            c_sems[j % 2],
        )

    c_hs = {0: codes_dma(0)}

    @pl.when(sid == 0)
    def _stage_shared():
        pltpu.sync_copy(rowsum_hbm, rowsum_sh)

    plsc.subcore_barrier()
    rs_h = pltpu.async_copy(rowsum_sh, rowsum_v, rs_sem)
    rs_h.wait()

    o_hs = {}
    for j in range(TPW):
        if j + 1 < TPW:
            c_hs[j + 1] = codes_dma(j + 1)
        c_hs[j].wait()
        if j >= 2:
            o_hs[j - 2].wait()

        def bg_body(bg, carry2):
            b16 = bg * 16
            for tt in range(TT):
                g = [
                    plsc.load_gather(
                        rowsum_v, [codes_v[j % 2, q, tt, pl.ds(b16, 16)]]
                    )
                    for q in range(Q)
                ]
                while len(g) > 1:
                    g = [a + b for a, b in zip(g[::2], g[1::2])]
                off = ((tbs[j] * TT + tt) * UPSAMPLE) * 0.001
                val = g[0] + off
                for k in range(UPSAMPLE):
                    out_v[j % 2, UPSAMPLE * tt + k, pl.ds(b16, 16)] = val
                    if k + 1 < UPSAMPLE:
                        val = val + 0.001
            return carry2

        lax.fori_loop(0, BB // 16, bg_body, 0)
        o_hs[j] = pltpu.async_copy(
            out_v.at[j % 2],
            out_hbm.at[pl.ds(tbs[j] * TT * UPSAMPLE, TT * UPSAMPLE),
                       pl.ds(cbs[j] * BB, BB)],
            o_sems[j % 2],
        )
    o_hs[TPW - 2].wait()
    o_hs[TPW - 1].wait()


@functools.partial(
    pl.kernel,
    out_type=jax.ShapeDtypeStruct((TOUT, B), jnp.float32),
    mesh=plsc.VectorSubcoreMesh(core_axis_name="c", subcore_axis_name="s"),
    scratch_types=[
        pltpu.VMEM((V,), jnp.float32),
        pltpu.VMEM((2, Q, TT, BB), jnp.int32),
        pltpu.VMEM((2, UPSAMPLE * TT, BB), jnp.float32),
        pltpu.VMEM_SHARED((V,), jnp.float32),
        pltpu.SemaphoreType.DMA,
        pltpu.SemaphoreType.DMA,
        pltpu.SemaphoreType.DMA,
        pltpu.SemaphoreType.DMA,
        pltpu.SemaphoreType.DMA,
    ],
    compiler_params=pltpu.CompilerParams(needs_layout_passes=False),
)
def _sc_gather(codes_hbm, rowsum_hbm, out_hbm, rowsum_v, codes_v, out_v,
               rowsum_sh, rs_sem, c_sem0, c_sem1, o_sem0, o_sem1):
    _sc_body(codes_hbm, rowsum_hbm, out_hbm, rowsum_v, codes_v, out_v,
             rowsum_sh, rs_sem, [c_sem0, c_sem1], [o_sem0, o_sem1])


def kernel(codes, table):
    rowsum = _rowsum_tc(table)
    out_t = _sc_gather(codes.transpose(1, 2, 0), rowsum)
    return out_t.T.reshape(B, 1, TOUT)


# speedup vs baseline: 1.1628x; 1.0049x over previous
import functools

import jax
import jax.numpy as jnp
from jax import lax
from jax.experimental import pallas as pl
from jax.experimental.pallas import tpu as pltpu
from jax.experimental.pallas import tpu_sc as plsc

UPSAMPLE = 4
V = 100000
H = 64
B, Q, T = 1024, 8, 200
TOUT = T * UPSAMPLE

RS_BLK = 25600
RS_GRID = -(-V // RS_BLK)

NW = 32
TT = 8
BB = 128
NTASK = (T // TT) * (B // BB)
NCB = B // BB
TPW = 7


def _rowsum_body(x_ref, o_ref):
    o_ref[...] = jnp.sum(x_ref[...], axis=0)


def _rowsum_tc(table):
    return pl.pallas_call(
        _rowsum_body,
        grid=(RS_GRID,),
        in_specs=[pl.BlockSpec((H, RS_BLK), lambda i: (0, i))],
        out_specs=pl.BlockSpec((RS_BLK,), lambda i: (i,)),
        out_shape=jax.ShapeDtypeStruct((V,), jnp.float32),
    )(table.T)


def _sc_body(codes_hbm, rowsum_hbm, out_hbm, rowsum_v, codes_v, out_v,
             rowsum_sh, rs_sem, c_sems, o_sems):
    cid = lax.axis_index("c")
    sid = lax.axis_index("s")
    wid = sid * 2 + cid

    tk0 = (wid * (NTASK - TPW)) // (NW - 1)
    tbs = []
    cbs = []
    for j in range(TPW):
        tk = tk0 + j
        tbs.append(tk // NCB)
        cbs.append(tk % NCB)

    def codes_dma(j):
        return pltpu.async_copy(
            codes_hbm.at[:, pl.ds(tbs[j] * TT, TT), pl.ds(cbs[j] * BB, BB)],
            codes_v.at[j % 2],
            c_sems[j % 2],
        )

    c_hs = {0: codes_dma(0)}

    @pl.when(sid == 0)
    def _stage_shared():
        pltpu.sync_copy(rowsum_hbm, rowsum_sh)

    plsc.subcore_barrier()
    rs_h = pltpu.async_copy(rowsum_sh, rowsum_v, rs_sem)
    rs_h.wait()

    o_hs = {}
    for j in range(TPW):
        if j + 1 < TPW:
            c_hs[j + 1] = codes_dma(j + 1)
        c_hs[j].wait()
        if j >= 2:
            o_hs[j - 2].wait()

        def bg_body(bg, carry2):
            b16 = bg * 16
            for tt in range(TT):
                g = [
                    plsc.load_gather(
                        rowsum_v, [codes_v[j % 2, q, tt, pl.ds(b16, 16)]]
                    )
                    for q in range(Q)
                ]
                while len(g) > 1:
                    g = [a + b for a, b in zip(g[::2], g[1::2])]
                off = ((tbs[j] * TT + tt) * UPSAMPLE) * 0.001
                val = g[0] + off
                for k in range(UPSAMPLE):
                    out_v[j % 2, UPSAMPLE * tt + k, pl.ds(b16, 16)] = val
                    if k + 1 < UPSAMPLE:
                        val = val + 0.001
            return carry2

        lax.fori_loop(0, BB // 16, bg_body, 0)
        o_hs[j] = pltpu.async_copy(
            out_v.at[j % 2],
            out_hbm.at[pl.ds(tbs[j] * TT * UPSAMPLE, TT * UPSAMPLE),
                       pl.ds(cbs[j] * BB, BB)],
            o_sems[j % 2],
        )
    o_hs[TPW - 2].wait()
    o_hs[TPW - 1].wait()


@functools.partial(
    pl.kernel,
    out_type=jax.ShapeDtypeStruct((TOUT, B), jnp.float32),
    mesh=plsc.VectorSubcoreMesh(core_axis_name="c", subcore_axis_name="s"),
    scratch_types=[
        pltpu.VMEM((V,), jnp.float32),
        pltpu.VMEM((2, Q, TT, BB), jnp.int32),
        pltpu.VMEM((2, UPSAMPLE * TT, BB), jnp.float32),
        pltpu.VMEM_SHARED((V,), jnp.float32),
        pltpu.SemaphoreType.DMA,
        pltpu.SemaphoreType.DMA,
        pltpu.SemaphoreType.DMA,
        pltpu.SemaphoreType.DMA,
        pltpu.SemaphoreType.DMA,
    ],
    compiler_params=pltpu.CompilerParams(needs_layout_passes=False),
)
def _sc_gather(codes_hbm, rowsum_hbm, out_hbm, rowsum_v, codes_v, out_v,
               rowsum_sh, rs_sem, c_sem0, c_sem1, o_sem0, o_sem1):
    _sc_body(codes_hbm, rowsum_hbm, out_hbm, rowsum_v, codes_v, out_v,
             rowsum_sh, rs_sem, [c_sem0, c_sem1], [o_sem0, o_sem1])


def kernel(codes, table):
    rowsum = _rowsum_tc(table)
    out_t = _sc_gather(codes.transpose(1, 2, 0), rowsum)
    return out_t.T.reshape(B, 1, TOUT)
